# Initial kernel scaffold; baseline (speedup 1.0000x reference)
import jax
import jax.numpy as jnp
from jax.experimental import pallas as pl


def kernel(features, index):
    raise NotImplementedError("write your pallas kernel here")


# speedup vs baseline: 6.1571x; 6.1571x over previous
import jax
import jax.numpy as jnp
from jax import lax
from jax.experimental import pallas as pl
from jax.experimental.pallas import tpu as pltpu
from jax.experimental.pallas import tpu_sc as plsc

_NUM_SEGMENTS = 10000
_N_EDGES = 320000
_D = 128
_NV = 64
_SPW = 160
_OUT_PAD = _NV * _SPW
_K = 256
_KS = 128
_NCH = _N_EDGES // _K
_LBUF = 168
_CBUF = 176
_NB = 80


def _seg_mean_body(feat_hbm, idx_hbm, bnd_hbm, out_hbm,
                   fbuf0, fbuf1, ibr0, ibr1, ib2d0, ib2d1,
                   cnt, acc, bbuf,
                   fsem0, fsem1, isem0, isem1,
                   acc_sh):
    cid = lax.axis_index("c")
    sid = lax.axis_index("s")
    wid = sid * 2 + cid
    zero16 = jnp.zeros((16,), jnp.float32)
    lanes = lax.iota(jnp.int32, 16)

    fbufs = (fbuf0, fbuf1)
    ibrs = (ibr0, ibr1)
    ib2ds = (ib2d0, ib2d1)
    fsems = (fsem0, fsem1)
    isems = (isem0, isem1)

    pltpu.sync_copy(bnd_hbm, bbuf)

    def start_fetch(b, c):
        c = jnp.minimum(c, _NCH - 1)
        base = c * _K
        pltpu.async_copy(feat_hbm.at[pl.ds(base, _K)], fbufs[b], fsems[b])
        pltpu.async_copy(idx_hbm.at[pl.ds(base, _K)], ibrs[b], isems[b])

    def wait_fetch(b):
        pltpu.make_async_copy(feat_hbm.at[pl.ds(0, _K)], fbufs[b], fsems[b]).wait()
        pltpu.make_async_copy(idx_hbm.at[pl.ds(0, _K)], ibrs[b], isems[b]).wait()

    for phase in range(2):
        vw = wid + phase * 32
        s0 = vw * _SPW

        def zrow(r, carry):
            for dg in range(_D // 16):
                acc[r, pl.ds(dg * 16, 16)] = zero16
            return carry

        lax.fori_loop(0, _LBUF, zrow, 0)
        for g in range(_CBUF // 16):
            cnt[pl.ds(g * 16, 16)] = zero16

        pltpu.sync_copy(acc, acc_sh.at[sid])

        bpair = bbuf[pl.ds(vw, 16)]
        e_lo = bpair[0]
        e_hi = bpair[1]
        c_lo = e_lo // _K
        c_hi = (e_hi + (_K - 1)) // _K
        n = c_hi - c_lo
        n2 = (n + 1) // 2

        for b in range(2):
            start_fetch(b, c_lo + b)

        @pl.loop(0, n2)
        def _(t):
            for b in range(2):
                i = t * 2 + b
                wait_fetch(b)

                @pl.when(i < n)
                def _():
                    for j in range(_K // 16):
                        d = ibrs[b][pl.ds(j * 16, 16)] - s0
                        v = jnp.where(d < 0, _SPW,
                                      jnp.where(d >= _SPW, _SPW + 1, d))
                        ib2ds[b][j // 8, pl.ds((j % 8) * 16, 16)] = v
                        v_prev = v.at[jnp.maximum(lanes - 1, 0)].get(
                            mode="promise_in_bounds")
                        v_next = v.at[jnp.minimum(lanes + 1, 15)].get(
                            mode="promise_in_bounds")
                        is_first = (lanes == 0) | (v != v_prev)
                        is_last = (lanes == 15) | (v != v_next)
                        runstart = plsc.cummax(jnp.where(is_first, lanes, 0))
                        runlen = (lanes - runstart + 1).astype(jnp.float32)
                        plsc.addupdate_scatter(cnt, [v], runlen, mask=is_last)
                    for h in range(_K // _KS):
                        pltpu.sync_copy(
                            fbufs[b].at[pl.ds(h * _KS, _KS)],
                            acc_sh.at[sid].at[ib2ds[b].at[h]], add=True)

                start_fetch(b, c_lo + i + 2)

        for b in range(2):
            wait_fetch(b)

        pltpu.sync_copy(acc_sh.at[sid], acc)

        def mean_group(g, carry):
            cvec = jnp.maximum(cnt[pl.ds(g * 16, 16)], 1.0)
            for j in range(16):
                d = cvec[j]
                for dg in range(_D // 16):
                    acc[g * 16 + j, pl.ds(dg * 16, 16)] = (
                        acc[g * 16 + j, pl.ds(dg * 16, 16)] / d)
            return carry

        lax.fori_loop(0, _SPW // 16, mean_group, 0)

        pltpu.sync_copy(acc.at[pl.ds(0, _SPW)], out_hbm.at[pl.ds(s0, _SPW)])


def kernel(features, index):
    index = index.astype(jnp.int32)
    seg_starts = jnp.arange(0, _OUT_PAD + 1, _SPW, dtype=jnp.int32)
    bnd = jnp.searchsorted(index, seg_starts).astype(jnp.int32)
    bnd = jnp.pad(bnd, (0, _NB - (_NV + 1)))

    mesh = plsc.VectorSubcoreMesh(core_axis_name="c", subcore_axis_name="s")
    out = pl.kernel(
        _seg_mean_body,
        out_type=jax.ShapeDtypeStruct((_OUT_PAD, _D), jnp.float32),
        mesh=mesh,
        compiler_params=pltpu.CompilerParams(needs_layout_passes=False),
        scratch_types=[
            pltpu.VMEM((_K, _D), jnp.float32),
            pltpu.VMEM((_K, _D), jnp.float32),
            pltpu.VMEM((_K,), jnp.int32),
            pltpu.VMEM((_K,), jnp.int32),
            pltpu.VMEM((2, _KS), jnp.int32),
            pltpu.VMEM((2, _KS), jnp.int32),
            pltpu.VMEM((_CBUF,), jnp.float32),
            pltpu.VMEM((_LBUF, _D), jnp.float32),
            pltpu.VMEM((_NB,), jnp.int32),
            pltpu.SemaphoreType.DMA,
            pltpu.SemaphoreType.DMA,
            pltpu.SemaphoreType.DMA,
            pltpu.SemaphoreType.DMA,
            pltpu.VMEM_SHARED((16, _LBUF, _D), jnp.float32),
        ],
    )(features, index, bnd)
    return out[:_NUM_SEGMENTS]


# speedup vs baseline: 9.8800x; 1.6046x over previous
import jax
import jax.numpy as jnp
from jax import lax
from jax.experimental import pallas as pl
from jax.experimental.pallas import tpu as pltpu
from jax.experimental.pallas import tpu_sc as plsc

_NUM_SEGMENTS = 10000
_N_EDGES = 320000
_D = 128
_NV = 64
_SPW = 160
_K = 128
_KS = 128
_NCH = _N_EDGES // _K
_LBUF = 168
_CBUF = 176
_NSAMP = 2512
_TAIL = _NUM_SEGMENTS - (_NUM_SEGMENTS // _SPW) * _SPW
_BIG = 2 ** 30


def _seg_mean_body(feat_hbm, idx_hbm, samp_hbm, out_hbm,
                   fbuf0, fbuf1, fbuf2, ibr0, ibr1, ibr2,
                   ib2d0, ib2d1, ib2d2,
                   cnt, acc, sbuf,
                   fsem0, fsem1, fsem2, isem0, isem1, isem2,
                   ssem0, ssem1, ssem2,
                   acc_sh):
    cid = lax.axis_index("c")
    sid = lax.axis_index("s")
    wid = sid * 2 + cid
    zero16 = jnp.zeros((16,), jnp.float32)
    lanes = lax.iota(jnp.int32, 16)

    fbufs = (fbuf0, fbuf1, fbuf2)
    ibrs = (ibr0, ibr1, ibr2)
    ib2ds = (ib2d0, ib2d1, ib2d2)
    fsems = (fsem0, fsem1, fsem2)
    isems = (isem0, isem1, isem2)
    ssems = (ssem0, ssem1, ssem2)

    pltpu.sync_copy(samp_hbm, sbuf)

    def start_fetch(b, c):
        base = c * _K
        pltpu.async_copy(feat_hbm.at[pl.ds(base, _K)], fbufs[b], fsems[b])
        pltpu.async_copy(idx_hbm.at[pl.ds(base, _K)], ibrs[b], isems[b])

    def wait_fetch(b):
        pltpu.make_async_copy(feat_hbm.at[pl.ds(0, _K)], fbufs[b], fsems[b]).wait()
        pltpu.make_async_copy(idx_hbm.at[pl.ds(0, _K)], ibrs[b], isems[b]).wait()

    def scatter_start(b):
        for h in range(_K // _KS):
            pltpu.async_copy(fbufs[b].at[pl.ds(h * _KS, _KS)],
                             acc_sh.at[sid].at[ib2ds[b].at[h]],
                             ssems[b], add=True)

    def scatter_wait(b):
        for h in range(_K // _KS):
            pltpu.make_async_copy(fbufs[b].at[pl.ds(h * _KS, _KS)],
                                  acc_sh.at[sid].at[ib2ds[b].at[h]],
                                  ssems[b]).wait()

    def lane_sum(v):
        for sh in (8, 4, 2, 1):
            v = v + v.at[jnp.bitwise_xor(lanes, sh)].get(
                mode="promise_in_bounds")
        return v[0]

    for phase in range(2):
        vw = wid + phase * 32
        s0 = vw * _SPW

        def zrow(r, carry):
            for dg in range(_D // 16):
                acc[r, pl.ds(dg * 16, 16)] = zero16
            return carry

        lax.fori_loop(0, _LBUF, zrow, 0)
        for g in range(_CBUF // 16):
            cnt[pl.ds(g * 16, 16)] = zero16

        pltpu.sync_copy(acc, acc_sh.at[sid])

        def cgroup(g, carry):
            clo_v, chi_v = carry
            v = sbuf[pl.ds(g * 16, 16)]
            clo_v = clo_v + jnp.where(v < s0, 1, 0).astype(jnp.int32)
            chi_v = chi_v + jnp.where(v < s0 + _SPW, 1, 0).astype(jnp.int32)
            return clo_v, chi_v

        zi = jnp.zeros((16,), jnp.int32)
        clo_v, chi_v = lax.fori_loop(0, _NSAMP // 16, cgroup, (zi, zi))
        c_lo = jnp.maximum(lane_sum(clo_v) - 1, 0)
        c_hi = lane_sum(chi_v)
        n = c_hi - c_lo
        n2 = (n + 1) // 2

        for u in range(3):
            @pl.when(u < n)
            def _():
                start_fetch(u, c_lo + u)

        n3 = (n + 3) // 3

        @pl.loop(0, n3)
        def _(t):
            for u in range(3):
                i = t * 3 + u

                @pl.when(i < n)
                def _():
                    wait_fetch(u)
                    for j in range(_K // 16):
                        d = ibrs[u][pl.ds(j * 16, 16)] - s0
                        v = jnp.where(d < 0, _SPW,
                                      jnp.where(d >= _SPW, _SPW + 1, d))
                        ib2ds[u][j // 8, pl.ds((j % 8) * 16, 16)] = v
                        v_prev = v.at[jnp.maximum(lanes - 1, 0)].get(
                            mode="promise_in_bounds")
                        v_next = v.at[jnp.minimum(lanes + 1, 15)].get(
                            mode="promise_in_bounds")
                        is_first = (lanes == 0) | (v != v_prev)
                        is_last = (lanes == 15) | (v != v_next)
                        runstart = plsc.cummax(jnp.where(is_first, lanes, 0))
                        runlen = (lanes - runstart + 1).astype(jnp.float32)
                        plsc.addupdate_scatter(cnt, [v], runlen, mask=is_last)
                    scatter_start(u)

                @pl.when((i >= 1) & (i - 1 < n))
                def _():
                    scatter_wait((u + 2) % 3)

                @pl.when((i >= 1) & (i + 2 < n))
                def _():
                    start_fetch((u + 2) % 3, c_lo + i + 2)

        pltpu.sync_copy(acc_sh.at[sid], acc)

        def mean_group(g, carry):
            cvec = jnp.maximum(cnt[pl.ds(g * 16, 16)], 1.0)
            for j in range(16):
                d = cvec[j]
                for dg in range(_D // 16):
                    acc[g * 16 + j, pl.ds(dg * 16, 16)] = (
                        acc[g * 16 + j, pl.ds(dg * 16, 16)] / d)
            return carry

        lax.fori_loop(0, _SPW // 16, mean_group, 0)

        @pl.when(s0 + _SPW <= _NUM_SEGMENTS)
        def _():
            pltpu.sync_copy(acc.at[pl.ds(0, _SPW)],
                            out_hbm.at[pl.ds(s0, _SPW)])

        @pl.when((s0 < _NUM_SEGMENTS) & (s0 + _SPW > _NUM_SEGMENTS))
        def _():
            pltpu.sync_copy(acc.at[pl.ds(0, _TAIL)],
                            out_hbm.at[pl.ds(s0, _TAIL)])


def kernel(features, index):
    index = index.astype(jnp.int32)
    samp = jnp.full((_NSAMP,), _BIG, jnp.int32)
    samp = lax.dynamic_update_slice(samp, index[:: _K], (0,))

    mesh = plsc.VectorSubcoreMesh(core_axis_name="c", subcore_axis_name="s")
    out = pl.kernel(
        _seg_mean_body,
        out_type=jax.ShapeDtypeStruct((_NUM_SEGMENTS, _D), jnp.float32),
        mesh=mesh,
        compiler_params=pltpu.CompilerParams(needs_layout_passes=False),
        scratch_types=[
            pltpu.VMEM((_K, _D), jnp.float32),
            pltpu.VMEM((_K, _D), jnp.float32),
            pltpu.VMEM((_K, _D), jnp.float32),
            pltpu.VMEM((_K,), jnp.int32),
            pltpu.VMEM((_K,), jnp.int32),
            pltpu.VMEM((_K,), jnp.int32),
            pltpu.VMEM((2, _KS), jnp.int32),
            pltpu.VMEM((2, _KS), jnp.int32),
            pltpu.VMEM((2, _KS), jnp.int32),
            pltpu.VMEM((_CBUF,), jnp.float32),
            pltpu.VMEM((_LBUF, _D), jnp.float32),
            pltpu.VMEM((_NSAMP,), jnp.int32),
            pltpu.SemaphoreType.DMA,
            pltpu.SemaphoreType.DMA,
            pltpu.SemaphoreType.DMA,
            pltpu.SemaphoreType.DMA,
            pltpu.SemaphoreType.DMA,
            pltpu.SemaphoreType.DMA,
            pltpu.SemaphoreType.DMA,
            pltpu.SemaphoreType.DMA,
            pltpu.SemaphoreType.DMA,
            pltpu.VMEM_SHARED((16, _LBUF, _D), jnp.float32),
        ],
    )(features, index, samp)
    return out


# speedup vs baseline: 9.9454x; 1.0066x over previous
import jax
import jax.numpy as jnp
from jax import lax
from jax.experimental import pallas as pl
from jax.experimental.pallas import tpu as pltpu
from jax.experimental.pallas import tpu_sc as plsc

_NUM_SEGMENTS = 10000
_N_EDGES = 320000
_D = 128
_NV = 64
_SPW = 160
_K = 128
_KS = 128
_NCH = _N_EDGES // _K
_LBUF = 168
_CBUF = 176
_NSAMP = 2512
_TAIL = _NUM_SEGMENTS - (_NUM_SEGMENTS // _SPW) * _SPW
_BIG = 2 ** 30


def _seg_mean_body(feat_hbm, idx_hbm, samp_hbm, out_hbm,
                   fbuf0, fbuf1, fbuf2, ibr0, ibr1, ibr2,
                   ib2d0, ib2d1, ib2d2,
                   cnt, acc, sbuf,
                   fsem0, fsem1, fsem2, isem0, isem1, isem2,
                   ssem0, ssem1, ssem2,
                   acc_sh):
    cid = lax.axis_index("c")
    sid = lax.axis_index("s")
    wid = sid * 2 + cid
    zero16 = jnp.zeros((16,), jnp.float32)
    lanes = lax.iota(jnp.int32, 16)

    fbufs = (fbuf0, fbuf1, fbuf2)
    ibrs = (ibr0, ibr1, ibr2)
    ib2ds = (ib2d0, ib2d1, ib2d2)
    fsems = (fsem0, fsem1, fsem2)
    isems = (isem0, isem1, isem2)
    ssems = (ssem0, ssem1, ssem2)

    pltpu.sync_copy(samp_hbm, sbuf)

    def start_fetch(b, c):
        base = c * _K
        pltpu.async_copy(feat_hbm.at[pl.ds(base, _K)], fbufs[b], fsems[b])
        pltpu.async_copy(idx_hbm.at[pl.ds(base, _K)], ibrs[b], isems[b])

    def wait_fetch(b):
        pltpu.make_async_copy(feat_hbm.at[pl.ds(0, _K)], fbufs[b], fsems[b]).wait()
        pltpu.make_async_copy(idx_hbm.at[pl.ds(0, _K)], ibrs[b], isems[b]).wait()

    def scatter_start(b):
        for h in range(_K // _KS):
            pltpu.async_copy(fbufs[b].at[pl.ds(h * _KS, _KS)],
                             acc_sh.at[sid].at[ib2ds[b].at[h]],
                             ssems[b], add=True)

    def scatter_wait(b):
        for h in range(_K // _KS):
            pltpu.make_async_copy(fbufs[b].at[pl.ds(h * _KS, _KS)],
                                  acc_sh.at[sid].at[ib2ds[b].at[h]],
                                  ssems[b]).wait()

    def lane_sum(v):
        for sh in (8, 4, 2, 1):
            v = v + v.at[jnp.bitwise_xor(lanes, sh)].get(
                mode="promise_in_bounds")
        return v[0]

    s0s = (wid * _SPW, (wid + 32) * _SPW)

    def cgroup(g, carry):
        v = sbuf[pl.ds(g * 16, 16)]
        return (carry[0] + jnp.where(v < s0s[0], 1, 0).astype(jnp.int32),
                carry[1] + jnp.where(v < s0s[0] + _SPW, 1, 0).astype(jnp.int32),
                carry[2] + jnp.where(v < s0s[1], 1, 0).astype(jnp.int32),
                carry[3] + jnp.where(v < s0s[1] + _SPW, 1, 0).astype(jnp.int32))

    zi = jnp.zeros((16,), jnp.int32)
    cvs = lax.fori_loop(0, _NSAMP // 16, cgroup, (zi, zi, zi, zi))
    c_los = (jnp.maximum(lane_sum(cvs[0]) - 1, 0),
             jnp.maximum(lane_sum(cvs[2]) - 1, 0))
    ns = (lane_sum(cvs[1]) - c_los[0], lane_sum(cvs[3]) - c_los[1])

    for phase in range(2):
        s0 = s0s[phase]
        c_lo = c_los[phase]
        n = ns[phase]

        if phase == 0:
            for u in range(3):
                @pl.when(u < n)
                def _():
                    start_fetch(u, c_lo + u)

        def zrow(r, carry):
            for dg in range(_D // 16):
                acc[r, pl.ds(dg * 16, 16)] = zero16
            return carry

        lax.fori_loop(0, _LBUF, zrow, 0)
        for g in range(_CBUF // 16):
            cnt[pl.ds(g * 16, 16)] = zero16

        pltpu.sync_copy(acc, acc_sh.at[sid])

        n3 = (n + 3) // 3

        @pl.loop(0, n3)
        def _(t):
            for u in range(3):
                i = t * 3 + u

                @pl.when(i < n)
                def _():
                    wait_fetch(u)
                    for j in range(_K // 16):
                        d = ibrs[u][pl.ds(j * 16, 16)] - s0
                        v = jnp.where(d < 0, _SPW,
                                      jnp.where(d >= _SPW, _SPW + 1, d))
                        ib2ds[u][j // 8, pl.ds((j % 8) * 16, 16)] = v
                        v_prev = v.at[jnp.maximum(lanes - 1, 0)].get(
                            mode="promise_in_bounds")
                        v_next = v.at[jnp.minimum(lanes + 1, 15)].get(
                            mode="promise_in_bounds")
                        is_first = (lanes == 0) | (v != v_prev)
                        is_last = (lanes == 15) | (v != v_next)
                        runstart = plsc.cummax(jnp.where(is_first, lanes, 0))
                        runlen = (lanes - runstart + 1).astype(jnp.float32)
                        plsc.addupdate_scatter(cnt, [v], runlen, mask=is_last)
                    scatter_start(u)

                @pl.when((i >= 1) & (i - 1 < n))
                def _():
                    scatter_wait((u + 2) % 3)

                @pl.when((i >= 1) & (i + 2 < n))
                def _():
                    start_fetch((u + 2) % 3, c_lo + i + 2)

        if phase == 0:
            for u in range(3):
                @pl.when(u < ns[1])
                def _():
                    start_fetch(u, c_los[1] + u)

        pltpu.sync_copy(acc_sh.at[sid], acc)

        def mean_group(g, carry):
            cvec = jnp.maximum(cnt[pl.ds(g * 16, 16)], 1.0)
            for j in range(16):
                d = cvec[j]
                for dg in range(_D // 16):
                    acc[g * 16 + j, pl.ds(dg * 16, 16)] = (
                        acc[g * 16 + j, pl.ds(dg * 16, 16)] / d)
            return carry

        lax.fori_loop(0, _SPW // 16, mean_group, 0)

        @pl.when(s0 + _SPW <= _NUM_SEGMENTS)
        def _():
            pltpu.sync_copy(acc.at[pl.ds(0, _SPW)],
                            out_hbm.at[pl.ds(s0, _SPW)])

        @pl.when((s0 < _NUM_SEGMENTS) & (s0 + _SPW > _NUM_SEGMENTS))
        def _():
            pltpu.sync_copy(acc.at[pl.ds(0, _TAIL)],
                            out_hbm.at[pl.ds(s0, _TAIL)])


def kernel(features, index):
    index = index.astype(jnp.int32)
    samp = jnp.full((_NSAMP,), _BIG, jnp.int32)
    samp = lax.dynamic_update_slice(samp, index[:: _K], (0,))

    mesh = plsc.VectorSubcoreMesh(core_axis_name="c", subcore_axis_name="s")
    out = pl.kernel(
        _seg_mean_body,
        out_type=jax.ShapeDtypeStruct((_NUM_SEGMENTS, _D), jnp.float32),
        mesh=mesh,
        compiler_params=pltpu.CompilerParams(needs_layout_passes=False),
        scratch_types=[
            pltpu.VMEM((_K, _D), jnp.float32),
            pltpu.VMEM((_K, _D), jnp.float32),
            pltpu.VMEM((_K, _D), jnp.float32),
            pltpu.VMEM((_K,), jnp.int32),
            pltpu.VMEM((_K,), jnp.int32),
            pltpu.VMEM((_K,), jnp.int32),
            pltpu.VMEM((2, _KS), jnp.int32),
            pltpu.VMEM((2, _KS), jnp.int32),
            pltpu.VMEM((2, _KS), jnp.int32),
            pltpu.VMEM((_CBUF,), jnp.float32),
            pltpu.VMEM((_LBUF, _D), jnp.float32),
            pltpu.VMEM((_NSAMP,), jnp.int32),
            pltpu.SemaphoreType.DMA,
            pltpu.SemaphoreType.DMA,
            pltpu.SemaphoreType.DMA,
            pltpu.SemaphoreType.DMA,
            pltpu.SemaphoreType.DMA,
            pltpu.SemaphoreType.DMA,
            pltpu.SemaphoreType.DMA,
            pltpu.SemaphoreType.DMA,
            pltpu.SemaphoreType.DMA,
            pltpu.VMEM_SHARED((16, _LBUF, _D), jnp.float32),
        ],
    )(features, index, samp)
    return out
